# baseline (speedup 1.0000x reference)
import jax
import jax.numpy as jnp
from jax import lax
from jax.experimental import pallas as pl
from jax.experimental.pallas import tpu as pltpu
from jax.experimental.pallas import tpu_sc as plsc

N_NODES = 10000
N_EDGES = 160000
HID = 32
D_EDGE = 16

NC = 2
NS = 16
NW = NC * NS

CH = 125
EPW = N_EDGES // NW
NCHUNK_W = EPW // CH
GRP = 8
GRP_ROWS = GRP * CH
NGRP = NCHUNK_W // GRP
NGRP_ALL = NW * NGRP
EF4_R = GRP_ROWS // 4
NPR = N_EDGES * HID // 128
NPN = N_NODES * HID // 128
RPT = 624
TAIL_BASE = NS * RPT
TAIL = N_NODES - TAIL_BASE


def _sc_mesh():
    return plsc.VectorSubcoreMesh(
        core_axis_name="c", subcore_axis_name="s", num_cores=NC, num_subcores=NS)


def _gather_body(nf_hbm, src_hbm, eft_hbm, out_hbm, ef4_hbm,
                 idx_v, rows_v, efslab, ef4b, sem, sem_wb):
    c = lax.axis_index("c")
    s = lax.axis_index("s")
    wid = s * NC + c
    pltpu.sync_copy(src_hbm.at[pl.ds(wid * NCHUNK_W, NCHUNK_W)], idx_v)
    d_iota = lax.iota(jnp.int32, 16)

    for g in range(NGRP):
        buf = g % 2
        if g >= 2:
            pltpu.make_async_copy(
                rows_v.at[buf],
                out_hbm.at[pl.ds(wid * EPW + (g - 2) * GRP_ROWS, GRP_ROWS)],
                sem_wb).wait()
        cps = [pltpu.async_copy(nf_hbm.at[idx_v.at[g * GRP + k]],
                                rows_v.at[buf].at[pl.ds(k * CH, CH)], sem)
               for k in range(GRP)]
        pltpu.sync_copy(
            eft_hbm.at[:, pl.ds(wid * EPW + g * GRP_ROWS, GRP_ROWS)], efslab)

        def repack(r, carry2):
            for q in range(4):
                v = plsc.load_gather(
                    efslab, [d_iota, jnp.full((16,), 4 * r + q, jnp.int32)])
                ef4b[r, pl.ds(16 * q, 16)] = v
                ef4b[r, pl.ds(64 + 16 * q, 16)] = v
            return carry2

        lax.fori_loop(0, EF4_R, repack, 0)
        pltpu.sync_copy(ef4b, ef4_hbm.at[wid * NGRP + g])
        for cp in cps:
            cp.wait()
        pltpu.async_copy(rows_v.at[buf],
                         out_hbm.at[pl.ds(wid * EPW + g * GRP_ROWS, GRP_ROWS)],
                         sem_wb)
    for g in range(NGRP - 2, NGRP):
        pltpu.make_async_copy(
            rows_v.at[g % 2],
            out_hbm.at[pl.ds(wid * EPW + g * GRP_ROWS, GRP_ROWS)],
            sem_wb).wait()


@jax.jit
def _gather(nf, src2d, eft):
    return pl.kernel(
        _gather_body,
        out_type=(jax.ShapeDtypeStruct((N_EDGES, HID), jnp.float32),
                  jax.ShapeDtypeStruct((NGRP_ALL, EF4_R, 128), jnp.float32)),
        mesh=_sc_mesh(),
        compiler_params=pltpu.CompilerParams(use_tc_tiling_on_sc=False,
                                             needs_layout_passes=False),
        scratch_types=[
            pltpu.VMEM((NCHUNK_W, CH), jnp.int32),
            pltpu.VMEM((2, GRP_ROWS, HID), jnp.float32),
            pltpu.VMEM((D_EDGE, GRP_ROWS), jnp.float32),
            pltpu.VMEM((EF4_R, 128), jnp.float32),
            pltpu.SemaphoreType.DMA,
            pltpu.SemaphoreType.DMA,
        ],
    )(nf, src2d, eft)


_MM_BLK = 1000
_MM_G = _MM_BLK // EF4_R


def _mm_body(x_ref, ef_ref, w_ref, r_ref, b_ref, m_ref):
    x = x_ref[...]
    ef = ef_ref[...].reshape(_MM_BLK, 128)
    acc = jnp.dot(x, b_ref[...], preferred_element_type=jnp.float32)
    for p in range(D_EDGE // 2):
        sp = jnp.dot(ef, r_ref[p], preferred_element_type=jnp.float32)
        zz = jnp.concatenate([sp[:, :128] * x, sp[:, 128:] * x], axis=1)
        acc += jnp.dot(zz, w_ref[p], preferred_element_type=jnp.float32)
    m_ref[...] = acc


@jax.jit
def _matmul(x4, ef4, w4p, rp, b4):
    return pl.pallas_call(
        _mm_body,
        grid=(NPR // _MM_BLK,),
        in_specs=[
            pl.BlockSpec((_MM_BLK, 128), lambda i: (i, 0)),
            pl.BlockSpec((_MM_G, EF4_R, 128), lambda i: (i, 0, 0)),
            pl.BlockSpec((D_EDGE // 2, 256, 128), lambda i: (0, 0, 0)),
            pl.BlockSpec((D_EDGE // 2, 128, 256), lambda i: (0, 0, 0)),
            pl.BlockSpec((128, 128), lambda i: (0, 0)),
        ],
        out_specs=pl.BlockSpec((_MM_BLK, 128), lambda i: (i, 0)),
        out_shape=jax.ShapeDtypeStruct((NPR, 128), jnp.float32),
    )(x4, ef4, w4p, rp, b4)


def _scatter_body(m_hbm, dst_hbm, zeros_hbm, part_hbm, idx_v, rows_v, agg_sh, sem):
    c = lax.axis_index("c")
    s = lax.axis_index("s")
    wid = s * NC + c
    pltpu.sync_copy(zeros_hbm.at[pl.ds(s * RPT, RPT)],
                    agg_sh.at[pl.ds(s * RPT, RPT)])

    @pl.when(s == 0)
    def _():
        pltpu.sync_copy(zeros_hbm.at[pl.ds(TAIL_BASE, TAIL)],
                        agg_sh.at[pl.ds(TAIL_BASE, TAIL)])

    pltpu.sync_copy(dst_hbm.at[pl.ds(wid * NCHUNK_W, NCHUNK_W)], idx_v)
    plsc.subcore_barrier()

    def group(g, carry):
        pltpu.sync_copy(m_hbm.at[pl.ds(wid * EPW + g * GRP_ROWS, GRP_ROWS)],
                        rows_v)
        for k in range(GRP):
            pltpu.sync_copy(rows_v.at[pl.ds(k * CH, CH)],
                            agg_sh.at[idx_v.at[g * GRP + k]], add=True)
        return carry

    lax.fori_loop(0, NGRP, group, 0)
    plsc.subcore_barrier()
    pltpu.sync_copy(agg_sh.at[pl.ds(s * RPT, RPT)],
                    part_hbm.at[c].at[pl.ds(s * RPT, RPT)])

    @pl.when(s == 0)
    def _():
        pltpu.sync_copy(agg_sh.at[pl.ds(TAIL_BASE, TAIL)],
                        part_hbm.at[c].at[pl.ds(TAIL_BASE, TAIL)])


@jax.jit
def _scatter(m, dst2d, zeros):
    return pl.kernel(
        _scatter_body,
        out_type=jax.ShapeDtypeStruct((NC, N_NODES, HID), jnp.float32),
        mesh=_sc_mesh(),
        compiler_params=pltpu.CompilerParams(use_tc_tiling_on_sc=False),
        scratch_types=[
            pltpu.VMEM((NCHUNK_W, CH), jnp.int32),
            pltpu.VMEM((GRP_ROWS, HID), jnp.float32),
            pltpu.VMEM_SHARED((N_NODES, HID), jnp.float32),
            pltpu.SemaphoreType.DMA,
        ],
    )(m, dst2d, zeros)


def _comb_body(p_ref, nf_ref, b_ref, o_ref):
    o_ref[...] = p_ref[0] + p_ref[1] + nf_ref[...] + b_ref[...]


@jax.jit
def _combine(part4, nf4, bias4):
    return pl.pallas_call(
        _comb_body,
        out_shape=jax.ShapeDtypeStruct((NPN, 128), jnp.float32),
    )(part4, nf4, bias4)


def kernel(nf, edge_index, initial_ef, W_edge, b_edge, bias):
    src2d = edge_index[0].astype(jnp.int32).reshape(N_EDGES // CH, CH)
    dst2d = edge_index[1].astype(jnp.int32).reshape(N_EDGES // CH, CH)

    w3 = W_edge.reshape(D_EDGE, HID, HID)
    eye4 = jnp.eye(4, dtype=jnp.float32)
    w4 = jnp.einsum('qr,dio->dqiro', eye4, w3).reshape(D_EDGE, 128, 128)
    w4p = w4.reshape(D_EDGE // 2, 256, 128)
    b4 = jnp.einsum('qr,io->qiro', eye4,
                    b_edge.reshape(HID, HID)).reshape(128, 128)
    j_ids = jnp.arange(128)[None, :, None]
    l_ids = jnp.arange(256)[None, None, :]
    p_ids = jnp.arange(D_EDGE // 2)[:, None, None]
    d_ids = 2 * p_ids + l_ids // 128
    rp = ((j_ids < 64) &
          (j_ids == 16 * ((l_ids % 128) // 32) + d_ids)).astype(jnp.float32)

    eft = initial_ef.T
    nf4 = nf.reshape(NPN, 128)
    zeros = jnp.zeros((N_NODES, HID), jnp.float32)
    bias4 = jnp.tile(bias, 4).reshape(1, 128)

    x_src, ef4 = _gather(nf, src2d, eft)
    x4 = x_src.reshape(NPR, 128)
    m = _matmul(x4, ef4, w4p, rp, b4).reshape(N_EDGES, HID)
    part4 = _scatter(m, dst2d, zeros).reshape(NC, NPN, 128)
    out4 = _combine(part4, nf4, bias4)
    return out4.reshape(N_NODES, HID)


# speedup vs baseline: 1.0826x; 1.0826x over previous
import jax
import jax.numpy as jnp
from jax import lax
from jax.experimental import pallas as pl
from jax.experimental.pallas import tpu as pltpu
from jax.experimental.pallas import tpu_sc as plsc

N_NODES = 10000
N_EDGES = 160000
HID = 32
D_EDGE = 16

NC = 2
NS = 16
NW = NC * NS

CH = 125
EPW = N_EDGES // NW
NCHUNK_W = EPW // CH
GRP = 8
GRP_ROWS = GRP * CH
NGRP = NCHUNK_W // GRP
NGRP_ALL = NW * NGRP
EF4_R = GRP_ROWS // 4
NPR = N_EDGES * HID // 128
NPN = N_NODES * HID // 128
RPT = 624
TAIL_BASE = NS * RPT
TAIL = N_NODES - TAIL_BASE


def _sc_mesh():
    return plsc.VectorSubcoreMesh(
        core_axis_name="c", subcore_axis_name="s", num_cores=NC, num_subcores=NS)


def _gather_body(nf_hbm, src_hbm, eft_hbm, out_hbm, ef4_hbm,
                 idx_v, rows_v, efslab, ef4b, sem, sem_wb):
    c = lax.axis_index("c")
    s = lax.axis_index("s")
    wid = s * NC + c
    pltpu.sync_copy(src_hbm.at[pl.ds(wid * NCHUNK_W, NCHUNK_W)], idx_v)
    d_iota = lax.iota(jnp.int32, 16)

    for g in range(NGRP):
        buf = g % 2
        if g >= 2:
            pltpu.make_async_copy(
                rows_v.at[buf],
                out_hbm.at[pl.ds(wid * EPW + (g - 2) * GRP_ROWS, GRP_ROWS)],
                sem_wb).wait()
        cps = [pltpu.async_copy(nf_hbm.at[idx_v.at[g * GRP + k]],
                                rows_v.at[buf].at[pl.ds(k * CH, CH)], sem)
               for k in range(GRP)]
        pltpu.sync_copy(
            eft_hbm.at[:, pl.ds(wid * EPW + g * GRP_ROWS, GRP_ROWS)], efslab)

        def repack(r, carry2):
            for q in range(4):
                v = plsc.load_gather(
                    efslab, [d_iota, jnp.full((16,), 4 * r + q, jnp.int32)])
                ef4b[r, pl.ds(16 * q, 16)] = v
            return carry2

        lax.fori_loop(0, EF4_R, repack, 0)
        pltpu.sync_copy(ef4b, ef4_hbm.at[wid * NGRP + g])
        for cp in cps:
            cp.wait()
        pltpu.async_copy(rows_v.at[buf],
                         out_hbm.at[pl.ds(wid * EPW + g * GRP_ROWS, GRP_ROWS)],
                         sem_wb)
    for g in range(NGRP - 2, NGRP):
        pltpu.make_async_copy(
            rows_v.at[g % 2],
            out_hbm.at[pl.ds(wid * EPW + g * GRP_ROWS, GRP_ROWS)],
            sem_wb).wait()


@jax.jit
def _gather(nf, src2d, eft):
    return pl.kernel(
        _gather_body,
        out_type=(jax.ShapeDtypeStruct((N_EDGES, HID), jnp.float32),
                  jax.ShapeDtypeStruct((NGRP_ALL, EF4_R, 64), jnp.float32)),
        mesh=_sc_mesh(),
        compiler_params=pltpu.CompilerParams(use_tc_tiling_on_sc=False,
                                             needs_layout_passes=False),
        scratch_types=[
            pltpu.VMEM((NCHUNK_W, CH), jnp.int32),
            pltpu.VMEM((2, GRP_ROWS, HID), jnp.float32),
            pltpu.VMEM((D_EDGE, GRP_ROWS), jnp.float32),
            pltpu.VMEM((EF4_R, 64), jnp.float32),
            pltpu.SemaphoreType.DMA,
            pltpu.SemaphoreType.DMA,
        ],
    )(nf, src2d, eft)


_MM_BLK = 2000
_MM_G = _MM_BLK // EF4_R


def _mm_body(x_ref, ef_ref, w_ref, r_ref, b_ref, m_ref):
    x = x_ref[...]
    ef = ef_ref[...].reshape(_MM_BLK, 64)
    acc = jnp.dot(x, b_ref[...], preferred_element_type=jnp.float32)
    for p in range(D_EDGE // 2):
        sp = jnp.dot(ef, r_ref[p], preferred_element_type=jnp.float32)
        zz = jnp.concatenate([sp[:, :128] * x, sp[:, 128:] * x], axis=1)
        acc += jnp.dot(zz, w_ref[p], preferred_element_type=jnp.float32)
    m_ref[...] = acc


@jax.jit
def _matmul(x4, ef4, w4p, rp, b4):
    return pl.pallas_call(
        _mm_body,
        grid=(NPR // _MM_BLK,),
        in_specs=[
            pl.BlockSpec((_MM_BLK, 128), lambda i: (i, 0)),
            pl.BlockSpec((_MM_G, EF4_R, 64), lambda i: (i, 0, 0)),
            pl.BlockSpec((D_EDGE // 2, 256, 128), lambda i: (0, 0, 0)),
            pl.BlockSpec((D_EDGE // 2, 64, 256), lambda i: (0, 0, 0)),
            pl.BlockSpec((128, 128), lambda i: (0, 0)),
        ],
        out_specs=pl.BlockSpec((_MM_BLK, 128), lambda i: (i, 0)),
        out_shape=jax.ShapeDtypeStruct((NPR, 128), jnp.float32),
    )(x4, ef4, w4p, rp, b4)


def _scatter_body(m_hbm, dst_hbm, zeros_hbm, part_hbm, idx_v, rows_v, agg_sh, sem):
    c = lax.axis_index("c")
    s = lax.axis_index("s")
    wid = s * NC + c
    pltpu.sync_copy(zeros_hbm.at[pl.ds(s * RPT, RPT)],
                    agg_sh.at[pl.ds(s * RPT, RPT)])

    @pl.when(s == 0)
    def _():
        pltpu.sync_copy(zeros_hbm.at[pl.ds(TAIL_BASE, TAIL)],
                        agg_sh.at[pl.ds(TAIL_BASE, TAIL)])

    pltpu.sync_copy(dst_hbm.at[pl.ds(wid * NCHUNK_W, NCHUNK_W)], idx_v)
    plsc.subcore_barrier()

    def group(g, carry):
        pltpu.sync_copy(m_hbm.at[pl.ds(wid * EPW + g * GRP_ROWS, GRP_ROWS)],
                        rows_v)
        for k in range(GRP):
            pltpu.sync_copy(rows_v.at[pl.ds(k * CH, CH)],
                            agg_sh.at[idx_v.at[g * GRP + k]], add=True)
        return carry

    lax.fori_loop(0, NGRP, group, 0)
    plsc.subcore_barrier()
    pltpu.sync_copy(agg_sh.at[pl.ds(s * RPT, RPT)],
                    part_hbm.at[c].at[pl.ds(s * RPT, RPT)])

    @pl.when(s == 0)
    def _():
        pltpu.sync_copy(agg_sh.at[pl.ds(TAIL_BASE, TAIL)],
                        part_hbm.at[c].at[pl.ds(TAIL_BASE, TAIL)])


@jax.jit
def _scatter(m, dst2d, zeros):
    return pl.kernel(
        _scatter_body,
        out_type=jax.ShapeDtypeStruct((NC, N_NODES, HID), jnp.float32),
        mesh=_sc_mesh(),
        compiler_params=pltpu.CompilerParams(use_tc_tiling_on_sc=False),
        scratch_types=[
            pltpu.VMEM((NCHUNK_W, CH), jnp.int32),
            pltpu.VMEM((GRP_ROWS, HID), jnp.float32),
            pltpu.VMEM_SHARED((N_NODES, HID), jnp.float32),
            pltpu.SemaphoreType.DMA,
        ],
    )(m, dst2d, zeros)


def _comb_body(p_ref, nf_ref, b_ref, o_ref):
    o_ref[...] = p_ref[0] + p_ref[1] + nf_ref[...] + b_ref[...]


@jax.jit
def _combine(part4, nf4, bias4):
    return pl.pallas_call(
        _comb_body,
        out_shape=jax.ShapeDtypeStruct((NPN, 128), jnp.float32),
    )(part4, nf4, bias4)


def kernel(nf, edge_index, initial_ef, W_edge, b_edge, bias):
    src2d = edge_index[0].astype(jnp.int32).reshape(N_EDGES // CH, CH)
    dst2d = edge_index[1].astype(jnp.int32).reshape(N_EDGES // CH, CH)

    w3 = W_edge.reshape(D_EDGE, HID, HID)
    eye4 = jnp.eye(4, dtype=jnp.float32)
    w4 = jnp.einsum('qr,dio->dqiro', eye4, w3).reshape(D_EDGE, 128, 128)
    w4p = w4.reshape(D_EDGE // 2, 256, 128)
    b4 = jnp.einsum('qr,io->qiro', eye4,
                    b_edge.reshape(HID, HID)).reshape(128, 128)
    j_ids = jnp.arange(64)[None, :, None]
    l_ids = jnp.arange(256)[None, None, :]
    p_ids = jnp.arange(D_EDGE // 2)[:, None, None]
    d_ids = 2 * p_ids + l_ids // 128
    rp = (j_ids == 16 * ((l_ids % 128) // 32) + d_ids).astype(jnp.float32)

    eft = initial_ef.T
    nf4 = nf.reshape(NPN, 128)
    zeros = jnp.zeros((N_NODES, HID), jnp.float32)
    bias4 = jnp.tile(bias, 4).reshape(1, 128)

    x_src, ef4 = _gather(nf, src2d, eft)
    x4 = x_src.reshape(NPR, 128)
    m = _matmul(x4, ef4, w4p, rp, b4).reshape(N_EDGES, HID)
    part4 = _scatter(m, dst2d, zeros).reshape(NC, NPN, 128)
    out4 = _combine(part4, nf4, bias4)
    return out4.reshape(N_NODES, HID)


# speedup vs baseline: 1.0883x; 1.0052x over previous
import jax
import jax.numpy as jnp
from jax import lax
from jax.experimental import pallas as pl
from jax.experimental.pallas import tpu as pltpu
from jax.experimental.pallas import tpu_sc as plsc

N_NODES = 10000
N_EDGES = 160000
HID = 32
D_EDGE = 16

NC = 2
NS = 16
NW = NC * NS

CH = 125
EPW = N_EDGES // NW
NCHUNK_W = EPW // CH
GRP = 8
GRP_ROWS = GRP * CH
NGRP = NCHUNK_W // GRP
NGRP_ALL = NW * NGRP
EF4_R = GRP_ROWS // 4
NPR = N_EDGES * HID // 128
NPN = N_NODES * HID // 128
RPT = 624
TAIL_BASE = NS * RPT
TAIL = N_NODES - TAIL_BASE


def _sc_mesh():
    return plsc.VectorSubcoreMesh(
        core_axis_name="c", subcore_axis_name="s", num_cores=NC, num_subcores=NS)


def _gather_body(nf_hbm, src_hbm, eft_hbm, out_hbm, ef4_hbm,
                 idx_v, rows_v, efslab, ef4b, sem, sem_wb):
    c = lax.axis_index("c")
    s = lax.axis_index("s")
    wid = s * NC + c
    pltpu.sync_copy(src_hbm.at[pl.ds(wid * NCHUNK_W, NCHUNK_W)], idx_v)
    d_iota = lax.iota(jnp.int32, 16)

    for g in range(NGRP):
        buf = g % 2
        if g >= 2:
            pltpu.make_async_copy(
                rows_v.at[buf],
                out_hbm.at[pl.ds(wid * EPW + (g - 2) * GRP_ROWS, GRP_ROWS)],
                sem_wb).wait()
        cps = [pltpu.async_copy(nf_hbm.at[idx_v.at[g * GRP + k]],
                                rows_v.at[buf].at[pl.ds(k * CH, CH)], sem)
               for k in range(GRP)]
        pltpu.sync_copy(
            eft_hbm.at[:, pl.ds(wid * EPW + g * GRP_ROWS, GRP_ROWS)], efslab)

        def repack(r, carry2):
            for q in range(4):
                v = plsc.load_gather(
                    efslab, [d_iota, jnp.full((16,), 4 * r + q, jnp.int32)])
                ef4b[r, pl.ds(16 * q, 16)] = v
            return carry2

        lax.fori_loop(0, EF4_R, repack, 0)
        pltpu.sync_copy(ef4b, ef4_hbm.at[wid * NGRP + g])
        for cp in cps:
            cp.wait()
        pltpu.async_copy(rows_v.at[buf],
                         out_hbm.at[pl.ds(wid * EPW + g * GRP_ROWS, GRP_ROWS)],
                         sem_wb)
    for g in range(NGRP - 2, NGRP):
        pltpu.make_async_copy(
            rows_v.at[g % 2],
            out_hbm.at[pl.ds(wid * EPW + g * GRP_ROWS, GRP_ROWS)],
            sem_wb).wait()


@jax.jit
def _gather(nf, src2d, eft):
    return pl.kernel(
        _gather_body,
        out_type=(jax.ShapeDtypeStruct((N_EDGES, HID), jnp.float32),
                  jax.ShapeDtypeStruct((NGRP_ALL, EF4_R, 64), jnp.float32)),
        mesh=_sc_mesh(),
        compiler_params=pltpu.CompilerParams(use_tc_tiling_on_sc=False,
                                             needs_layout_passes=False),
        scratch_types=[
            pltpu.VMEM((NCHUNK_W, CH), jnp.int32),
            pltpu.VMEM((2, GRP_ROWS, HID), jnp.float32),
            pltpu.VMEM((D_EDGE, GRP_ROWS), jnp.float32),
            pltpu.VMEM((EF4_R, 64), jnp.float32),
            pltpu.SemaphoreType.DMA,
            pltpu.SemaphoreType.DMA,
        ],
    )(nf, src2d, eft)


_MM_BLK = 4000
_MM_G = _MM_BLK // EF4_R


def _mm_body(x_ref, ef_ref, w_ref, r_ref, b_ref, m_ref):
    x = x_ref[...]
    ef = ef_ref[...].reshape(_MM_BLK, 64)
    acc = jnp.dot(x, b_ref[...], preferred_element_type=jnp.float32)
    for p in range(D_EDGE // 2):
        sp = jnp.dot(ef, r_ref[p], preferred_element_type=jnp.float32)
        zz = jnp.concatenate([sp[:, :128] * x, sp[:, 128:] * x], axis=1)
        acc += jnp.dot(zz, w_ref[p], preferred_element_type=jnp.float32)
    m_ref[...] = acc


@jax.jit
def _matmul(x4, ef4, w4p, rp, b4):
    return pl.pallas_call(
        _mm_body,
        grid=(NPR // _MM_BLK,),
        in_specs=[
            pl.BlockSpec((_MM_BLK, 128), lambda i: (i, 0)),
            pl.BlockSpec((_MM_G, EF4_R, 64), lambda i: (i, 0, 0)),
            pl.BlockSpec((D_EDGE // 2, 256, 128), lambda i: (0, 0, 0)),
            pl.BlockSpec((D_EDGE // 2, 64, 256), lambda i: (0, 0, 0)),
            pl.BlockSpec((128, 128), lambda i: (0, 0)),
        ],
        out_specs=pl.BlockSpec((_MM_BLK, 128), lambda i: (i, 0)),
        out_shape=jax.ShapeDtypeStruct((NPR, 128), jnp.float32),
    )(x4, ef4, w4p, rp, b4)


def _scatter_body(m_hbm, dst_hbm, zeros_hbm, part_hbm, idx_v, rows_v, agg_sh, sem):
    c = lax.axis_index("c")
    s = lax.axis_index("s")
    wid = s * NC + c
    pltpu.sync_copy(zeros_hbm.at[pl.ds(s * RPT, RPT)],
                    agg_sh.at[pl.ds(s * RPT, RPT)])

    @pl.when(s == 0)
    def _():
        pltpu.sync_copy(zeros_hbm.at[pl.ds(TAIL_BASE, TAIL)],
                        agg_sh.at[pl.ds(TAIL_BASE, TAIL)])

    pltpu.sync_copy(dst_hbm.at[pl.ds(wid * NCHUNK_W, NCHUNK_W)], idx_v)
    plsc.subcore_barrier()

    def group(g, carry):
        pltpu.sync_copy(m_hbm.at[pl.ds(wid * EPW + g * GRP_ROWS, GRP_ROWS)],
                        rows_v)
        for k in range(GRP):
            pltpu.sync_copy(rows_v.at[pl.ds(k * CH, CH)],
                            agg_sh.at[idx_v.at[g * GRP + k]], add=True)
        return carry

    lax.fori_loop(0, NGRP, group, 0)
    plsc.subcore_barrier()
    pltpu.sync_copy(agg_sh.at[pl.ds(s * RPT, RPT)],
                    part_hbm.at[c].at[pl.ds(s * RPT, RPT)])

    @pl.when(s == 0)
    def _():
        pltpu.sync_copy(agg_sh.at[pl.ds(TAIL_BASE, TAIL)],
                        part_hbm.at[c].at[pl.ds(TAIL_BASE, TAIL)])


@jax.jit
def _scatter(m, dst2d, zeros):
    return pl.kernel(
        _scatter_body,
        out_type=jax.ShapeDtypeStruct((NC, N_NODES, HID), jnp.float32),
        mesh=_sc_mesh(),
        compiler_params=pltpu.CompilerParams(use_tc_tiling_on_sc=False),
        scratch_types=[
            pltpu.VMEM((NCHUNK_W, CH), jnp.int32),
            pltpu.VMEM((GRP_ROWS, HID), jnp.float32),
            pltpu.VMEM_SHARED((N_NODES, HID), jnp.float32),
            pltpu.SemaphoreType.DMA,
        ],
    )(m, dst2d, zeros)


def _comb_body(p_ref, nf_ref, b_ref, o_ref):
    o_ref[...] = p_ref[0] + p_ref[1] + nf_ref[...] + b_ref[...]


@jax.jit
def _combine(part4, nf4, bias4):
    return pl.pallas_call(
        _comb_body,
        out_shape=jax.ShapeDtypeStruct((NPN, 128), jnp.float32),
    )(part4, nf4, bias4)


def kernel(nf, edge_index, initial_ef, W_edge, b_edge, bias):
    src2d = edge_index[0].astype(jnp.int32).reshape(N_EDGES // CH, CH)
    dst2d = edge_index[1].astype(jnp.int32).reshape(N_EDGES // CH, CH)

    w3 = W_edge.reshape(D_EDGE, HID, HID)
    eye4 = jnp.eye(4, dtype=jnp.float32)
    w4 = jnp.einsum('qr,dio->dqiro', eye4, w3).reshape(D_EDGE, 128, 128)
    w4p = w4.reshape(D_EDGE // 2, 256, 128)
    b4 = jnp.einsum('qr,io->qiro', eye4,
                    b_edge.reshape(HID, HID)).reshape(128, 128)
    j_ids = jnp.arange(64)[None, :, None]
    l_ids = jnp.arange(256)[None, None, :]
    p_ids = jnp.arange(D_EDGE // 2)[:, None, None]
    d_ids = 2 * p_ids + l_ids // 128
    rp = (j_ids == 16 * ((l_ids % 128) // 32) + d_ids).astype(jnp.float32)

    eft = initial_ef.T
    nf4 = nf.reshape(NPN, 128)
    zeros = jnp.zeros((N_NODES, HID), jnp.float32)
    bias4 = jnp.tile(bias, 4).reshape(1, 128)

    x_src, ef4 = _gather(nf, src2d, eft)
    x4 = x_src.reshape(NPR, 128)
    m = _matmul(x4, ef4, w4p, rp, b4).reshape(N_EDGES, HID)
    part4 = _scatter(m, dst2d, zeros).reshape(NC, NPN, 128)
    out4 = _combine(part4, nf4, bias4)
    return out4.reshape(N_NODES, HID)


# speedup vs baseline: 1.1095x; 1.0195x over previous
import jax
import jax.numpy as jnp
from jax import lax
from jax.experimental import pallas as pl
from jax.experimental.pallas import tpu as pltpu
from jax.experimental.pallas import tpu_sc as plsc

N_NODES = 10000
N_EDGES = 160000
HID = 32
D_EDGE = 16

NC = 2
NS = 16
NW = NC * NS

CH = 125
EPW = N_EDGES // NW
NCHUNK_W = EPW // CH
GRP = 8
GRP_ROWS = GRP * CH
NGRP = NCHUNK_W // GRP
NGRP_ALL = NW * NGRP
EF4_R = GRP_ROWS // 4
NPR = N_EDGES * HID // 128
NPN = N_NODES * HID // 128
RPT = 624
TAIL_BASE = NS * RPT
TAIL = N_NODES - TAIL_BASE


def _sc_mesh():
    return plsc.VectorSubcoreMesh(
        core_axis_name="c", subcore_axis_name="s", num_cores=NC, num_subcores=NS)


def _gather_body(nf_hbm, src_hbm, eft_hbm, out_hbm, ef4_hbm,
                 idx_v, rows_v, efslab, ef4b, sem, sem_wb):
    c = lax.axis_index("c")
    s = lax.axis_index("s")
    wid = s * NC + c
    pltpu.sync_copy(src_hbm.at[pl.ds(wid * NCHUNK_W, NCHUNK_W)], idx_v)
    d_iota = lax.iota(jnp.int32, 16)

    for g in range(NGRP):
        buf = g % 2
        if g >= 2:
            pltpu.make_async_copy(
                rows_v.at[buf],
                out_hbm.at[pl.ds(wid * EPW + (g - 2) * GRP_ROWS, GRP_ROWS)],
                sem_wb).wait()
        cps = [pltpu.async_copy(nf_hbm.at[idx_v.at[g * GRP + k]],
                                rows_v.at[buf].at[pl.ds(k * CH, CH)], sem)
               for k in range(GRP)]
        pltpu.sync_copy(
            eft_hbm.at[:, pl.ds(wid * EPW + g * GRP_ROWS, GRP_ROWS)], efslab)

        def repack(r, carry2):
            for q in range(4):
                v = plsc.load_gather(
                    efslab, [d_iota, jnp.full((16,), 4 * r + q, jnp.int32)])
                ef4b[r, pl.ds(16 * q, 16)] = v
            return carry2

        lax.fori_loop(0, EF4_R, repack, 0)
        pltpu.sync_copy(ef4b, ef4_hbm.at[wid * NGRP + g])
        for cp in cps:
            cp.wait()
        pltpu.async_copy(rows_v.at[buf],
                         out_hbm.at[pl.ds(wid * EPW + g * GRP_ROWS, GRP_ROWS)],
                         sem_wb)
    for g in range(NGRP - 2, NGRP):
        pltpu.make_async_copy(
            rows_v.at[g % 2],
            out_hbm.at[pl.ds(wid * EPW + g * GRP_ROWS, GRP_ROWS)],
            sem_wb).wait()


@jax.jit
def _gather(nf, src2d, eft):
    return pl.kernel(
        _gather_body,
        out_type=(jax.ShapeDtypeStruct((N_EDGES, HID), jnp.float32),
                  jax.ShapeDtypeStruct((NGRP_ALL, EF4_R, 64), jnp.float32)),
        mesh=_sc_mesh(),
        compiler_params=pltpu.CompilerParams(use_tc_tiling_on_sc=False,
                                             needs_layout_passes=False),
        scratch_types=[
            pltpu.VMEM((NCHUNK_W, CH), jnp.int32),
            pltpu.VMEM((2, GRP_ROWS, HID), jnp.float32),
            pltpu.VMEM((D_EDGE, GRP_ROWS), jnp.float32),
            pltpu.VMEM((EF4_R, 64), jnp.float32),
            pltpu.SemaphoreType.DMA,
            pltpu.SemaphoreType.DMA,
        ],
    )(nf, src2d, eft)


_MM_BLK = 4000
_MM_G = _MM_BLK // EF4_R


def _mm_body(x_ref, ef_ref, w_ref, r_ref, b_ref, m_ref):
    x = x_ref[...]
    ef = ef_ref[...].reshape(_MM_BLK, 64)
    acc = jnp.dot(x, b_ref[...], preferred_element_type=jnp.float32)
    for p in range(D_EDGE // 2):
        sp = jnp.dot(ef, r_ref[p], preferred_element_type=jnp.float32)
        zz = jnp.concatenate([sp[:, :128] * x, sp[:, 128:] * x], axis=1)
        acc += jnp.dot(zz, w_ref[p], preferred_element_type=jnp.float32)
    m_ref[...] = acc


@jax.jit
def _matmul(x4, ef4, w4p, rp, b4):
    return pl.pallas_call(
        _mm_body,
        grid=(NPR // _MM_BLK,),
        in_specs=[
            pl.BlockSpec((_MM_BLK, 128), lambda i: (i, 0)),
            pl.BlockSpec((_MM_G, EF4_R, 64), lambda i: (i, 0, 0)),
            pl.BlockSpec((D_EDGE // 2, 256, 128), lambda i: (0, 0, 0)),
            pl.BlockSpec((D_EDGE // 2, 64, 256), lambda i: (0, 0, 0)),
            pl.BlockSpec((128, 128), lambda i: (0, 0)),
        ],
        out_specs=pl.BlockSpec((_MM_BLK, 128), lambda i: (i, 0)),
        out_shape=jax.ShapeDtypeStruct((NPR, 128), jnp.float32),
    )(x4, ef4, w4p, rp, b4)


def _scatter_body(m_hbm, dst_hbm, zeros_hbm, part_hbm, idx_v, rows_v, agg_sh,
                  sem_ld):
    c = lax.axis_index("c")
    s = lax.axis_index("s")
    wid = s * NC + c
    pltpu.sync_copy(zeros_hbm.at[pl.ds(s * RPT, RPT)],
                    agg_sh.at[pl.ds(s * RPT, RPT)])

    @pl.when(s == 0)
    def _():
        pltpu.sync_copy(zeros_hbm.at[pl.ds(TAIL_BASE, TAIL)],
                        agg_sh.at[pl.ds(TAIL_BASE, TAIL)])

    pltpu.sync_copy(dst_hbm.at[pl.ds(wid * NCHUNK_W, NCHUNK_W)], idx_v)
    plsc.subcore_barrier()

    pltpu.async_copy(m_hbm.at[pl.ds(wid * EPW, GRP_ROWS)],
                     rows_v.at[0], sem_ld)
    for g in range(NGRP):
        buf = g % 2
        pltpu.make_async_copy(
            m_hbm.at[pl.ds(wid * EPW + g * GRP_ROWS, GRP_ROWS)],
            rows_v.at[buf], sem_ld).wait()
        if g + 1 < NGRP:
            pltpu.async_copy(
                m_hbm.at[pl.ds(wid * EPW + (g + 1) * GRP_ROWS, GRP_ROWS)],
                rows_v.at[1 - buf], sem_ld)
        for k in range(GRP):
            pltpu.sync_copy(rows_v.at[buf].at[pl.ds(k * CH, CH)],
                            agg_sh.at[idx_v.at[g * GRP + k]], add=True)
    plsc.subcore_barrier()
    pltpu.sync_copy(agg_sh.at[pl.ds(s * RPT, RPT)],
                    part_hbm.at[c].at[pl.ds(s * RPT, RPT)])

    @pl.when(s == 0)
    def _():
        pltpu.sync_copy(agg_sh.at[pl.ds(TAIL_BASE, TAIL)],
                        part_hbm.at[c].at[pl.ds(TAIL_BASE, TAIL)])


@jax.jit
def _scatter(m, dst2d, zeros):
    return pl.kernel(
        _scatter_body,
        out_type=jax.ShapeDtypeStruct((NC, N_NODES, HID), jnp.float32),
        mesh=_sc_mesh(),
        compiler_params=pltpu.CompilerParams(use_tc_tiling_on_sc=False),
        scratch_types=[
            pltpu.VMEM((NCHUNK_W, CH), jnp.int32),
            pltpu.VMEM((2, GRP_ROWS, HID), jnp.float32),
            pltpu.VMEM_SHARED((N_NODES, HID), jnp.float32),
            pltpu.SemaphoreType.DMA,
        ],
    )(m, dst2d, zeros)


def _comb_body(p_ref, nf_ref, b_ref, o_ref):
    o_ref[...] = p_ref[0] + p_ref[1] + nf_ref[...] + b_ref[...]


@jax.jit
def _combine(part4, nf4, bias4):
    return pl.pallas_call(
        _comb_body,
        out_shape=jax.ShapeDtypeStruct((NPN, 128), jnp.float32),
    )(part4, nf4, bias4)


def kernel(nf, edge_index, initial_ef, W_edge, b_edge, bias):
    src2d = edge_index[0].astype(jnp.int32).reshape(N_EDGES // CH, CH)
    dst2d = edge_index[1].astype(jnp.int32).reshape(N_EDGES // CH, CH)

    w3 = W_edge.reshape(D_EDGE, HID, HID)
    eye4 = jnp.eye(4, dtype=jnp.float32)
    w4 = jnp.einsum('qr,dio->dqiro', eye4, w3).reshape(D_EDGE, 128, 128)
    w4p = w4.reshape(D_EDGE // 2, 256, 128)
    b4 = jnp.einsum('qr,io->qiro', eye4,
                    b_edge.reshape(HID, HID)).reshape(128, 128)
    j_ids = jnp.arange(64)[None, :, None]
    l_ids = jnp.arange(256)[None, None, :]
    p_ids = jnp.arange(D_EDGE // 2)[:, None, None]
    d_ids = 2 * p_ids + l_ids // 128
    rp = (j_ids == 16 * ((l_ids % 128) // 32) + d_ids).astype(jnp.float32)

    eft = initial_ef.T
    nf4 = nf.reshape(NPN, 128)
    zeros = jnp.zeros((N_NODES, HID), jnp.float32)
    bias4 = jnp.tile(bias, 4).reshape(1, 128)

    x_src, ef4 = _gather(nf, src2d, eft)
    x4 = x_src.reshape(NPR, 128)
    m = _matmul(x4, ef4, w4p, rp, b4).reshape(N_EDGES, HID)
    part4 = _scatter(m, dst2d, zeros).reshape(NC, NPN, 128)
    out4 = _combine(part4, nf4, bias4)
    return out4.reshape(N_NODES, HID)
